# baseline (speedup 1.0000x reference)
import functools

import jax
import jax.numpy as jnp
from jax import lax
from jax.experimental import pallas as pl
from jax.experimental.pallas import tpu as pltpu
from jax.experimental.pallas import tpu_sc as plsc

NC, NS, L = 2, 16, 16
NW = NC * NS

R = 16384
C = 26
CP = 128
B = R * C
D = 128
CHUNK = 4 * C
B_PER_W = B // NW
R_PER_W = R // NW
R_STAGE = 256
N_CHUNKS = B_PER_W // CHUNK
NBUF = 6
N_LOOP = N_CHUNKS // NBUF
TAIL = N_LOOP * NBUF


def _emb_body(table_hbm, fea_hbm, out_hbm, raw_v, idx_v, bufs, gsems, osems):
    wid = lax.axis_index("s") * NC + lax.axis_index("c")
    row_base = wid * B_PER_W

    for h in range(2):
        pltpu.sync_copy(
            fea_hbm.at[pl.ds(wid * R_PER_W + h * R_STAGE, R_STAGE)], raw_v)

        def repack(r, _, h=h):
            base = (h * R_STAGE + r) * C
            idx_v[pl.ds(base, 16)] = raw_v[r, pl.ds(0, 16)]
            idx_v[pl.ds(base + 10, 16)] = raw_v[r, pl.ds(10, 16)]
            return 0

        lax.fori_loop(0, R_STAGE, repack, 0)

    def gather(j, b):
        off = pl.multiple_of(j * CHUNK, 8)
        return pltpu.make_async_copy(
            table_hbm.at[idx_v.at[pl.ds(off, CHUNK)]], bufs[b], gsems[b])

    def writeback(j, b):
        return pltpu.make_async_copy(
            bufs[b], out_hbm.at[pl.ds(row_base + j * CHUNK, CHUNK)], osems[b])

    for b in range(NBUF):
        gather(b, b).start()

    def step(jj, _):
        for b in range(NBUF):
            j = jj * NBUF + b
            gather(j, b).wait()
            writeback(j, b).start()
            nxt = j + NBUF

            @pl.when(nxt < N_CHUNKS)
            def _():
                writeback(j, b).wait()
                gather(nxt, b).start()
        return 0

    lax.fori_loop(0, N_LOOP, step, 0)

    for j in range(TAIL, N_CHUNKS):
        gather(j, j % NBUF).wait()
        writeback(j, j % NBUF).start()
    for j in range(N_CHUNKS - NBUF, N_CHUNKS):
        writeback(j, j % NBUF).wait()


@jax.jit
def _emb_lookup(fea, table):
    mesh = plsc.VectorSubcoreMesh(core_axis_name="c", subcore_axis_name="s")
    f = pl.kernel(
        _emb_body,
        out_type=jax.ShapeDtypeStruct((B, D), jnp.float32),
        mesh=mesh,
        scratch_types=[
            pltpu.VMEM((R_STAGE, CP), jnp.int32),
            pltpu.VMEM((B_PER_W,), jnp.int32),
            [pltpu.VMEM((CHUNK, D), jnp.float32) for _ in range(NBUF)],
            [pltpu.SemaphoreType.DMA for _ in range(NBUF)],
            [pltpu.SemaphoreType.DMA for _ in range(NBUF)],
        ],
    )
    return f(table, fea)


def kernel(node_fea, table):
    fea = node_fea.astype(jnp.int32)
    fea = jnp.pad(fea, ((0, 0), (0, CP - C)))
    return _emb_lookup(fea, table)


# speedup vs baseline: 1.0194x; 1.0194x over previous
import functools

import jax
import jax.numpy as jnp
from jax import lax
from jax.experimental import pallas as pl
from jax.experimental.pallas import tpu as pltpu
from jax.experimental.pallas import tpu_sc as plsc

NC, NS, L = 2, 16, 16
NW = NC * NS

B = 16384 * 26
D = 128
CHUNK = 128
B_PER_W = B // NW
N_CHUNKS = B_PER_W // CHUNK
NBUF = 7
N_LOOP = N_CHUNKS // NBUF
TAIL = N_LOOP * NBUF


def _emb_body(table_hbm, idx_hbm, out_hbm, idx_v, bufs, gsems, osems):
    wid = lax.axis_index("s") * NC + lax.axis_index("c")
    row_base = wid * B_PER_W

    pltpu.sync_copy(idx_hbm.at[pl.ds(row_base, B_PER_W)], idx_v)

    def gather(j, b):
        off = pl.multiple_of(j * CHUNK, 8)
        return pltpu.make_async_copy(
            table_hbm.at[idx_v.at[pl.ds(off, CHUNK)]], bufs[b], gsems[b])

    def writeback(j, b):
        return pltpu.make_async_copy(
            bufs[b], out_hbm.at[pl.ds(row_base + j * CHUNK, CHUNK)], osems[b])

    for b in range(NBUF):
        gather(b, b).start()

    def step(jj, _):
        for b in range(NBUF):
            j = jj * NBUF + b
            gather(j, b).wait()
            writeback(j, b).start()
            nxt = j + NBUF

            @pl.when(nxt < N_CHUNKS)
            def _():
                writeback(j, b).wait()
                gather(nxt, b).start()
        return 0

    lax.fori_loop(0, N_LOOP, step, 0)

    for j in range(TAIL, N_CHUNKS):
        gather(j, j % NBUF).wait()
        writeback(j, j % NBUF).start()
    for j in range(N_CHUNKS - NBUF, N_CHUNKS):
        writeback(j, j % NBUF).wait()


@jax.jit
def _emb_lookup(idx_flat, table):
    mesh = plsc.VectorSubcoreMesh(core_axis_name="c", subcore_axis_name="s")
    f = pl.kernel(
        _emb_body,
        out_type=jax.ShapeDtypeStruct((B, D), jnp.float32),
        mesh=mesh,
        scratch_types=[
            pltpu.VMEM((B_PER_W,), jnp.int32),
            [pltpu.VMEM((CHUNK, D), jnp.float32) for _ in range(NBUF)],
            [pltpu.SemaphoreType.DMA for _ in range(NBUF)],
            [pltpu.SemaphoreType.DMA for _ in range(NBUF)],
        ],
    )
    return f(table, idx_flat)


def kernel(node_fea, table):
    idx_flat = node_fea.astype(jnp.int32).reshape(B)
    return _emb_lookup(idx_flat, table)


# speedup vs baseline: 1.0328x; 1.0131x over previous
import functools

import jax
import jax.numpy as jnp
from jax import lax
from jax.experimental import pallas as pl
from jax.experimental.pallas import tpu as pltpu
from jax.experimental.pallas import tpu_sc as plsc

NC, NS, L = 2, 16, 16
NW = NC * NS

B = 16384 * 26
D = 128
CHUNK = 104
B_PER_W = B // NW
N_CHUNKS = B_PER_W // CHUNK
NBUF = 8
N_LOOP = N_CHUNKS // NBUF
TAIL = N_LOOP * NBUF


def _emb_body(table_hbm, idx_hbm, out_hbm, idx_v, bufs, gsems, osems):
    wid = lax.axis_index("s") * NC + lax.axis_index("c")
    row_base = wid * B_PER_W

    pltpu.sync_copy(idx_hbm.at[wid], idx_v)

    def gather(j, b):
        return pltpu.make_async_copy(
            table_hbm.at[idx_v.at[j]], bufs[b], gsems[b])

    def writeback(j, b):
        return pltpu.make_async_copy(
            bufs[b], out_hbm.at[pl.ds(row_base + j * CHUNK, CHUNK)], osems[b])

    for b in range(NBUF):
        gather(b, b).start()

    def step(jj, _):
        for b in range(NBUF):
            j = jj * NBUF + b
            gather(j, b).wait()
            writeback(j, b).start()
            nxt = j + NBUF

            @pl.when(nxt < N_CHUNKS)
            def _():
                writeback(j, b).wait()
                gather(nxt, b).start()
        return 0

    lax.fori_loop(0, N_LOOP, step, 0)

    for j in range(TAIL, N_CHUNKS):
        gather(j, j % NBUF).wait()
        writeback(j, j % NBUF).start()
    for j in range(N_CHUNKS - NBUF, N_CHUNKS):
        writeback(j, j % NBUF).wait()


@jax.jit
def _emb_lookup(idx2d, table):
    mesh = plsc.VectorSubcoreMesh(core_axis_name="c", subcore_axis_name="s")
    f = pl.kernel(
        _emb_body,
        out_type=jax.ShapeDtypeStruct((B, D), jnp.float32),
        mesh=mesh,
        scratch_types=[
            pltpu.VMEM((N_CHUNKS, CHUNK), jnp.int32),
            [pltpu.VMEM((CHUNK, D), jnp.float32) for _ in range(NBUF)],
            [pltpu.SemaphoreType.DMA for _ in range(NBUF)],
            [pltpu.SemaphoreType.DMA for _ in range(NBUF)],
        ],
    )
    return f(table, idx2d)


def kernel(node_fea, table):
    idx2d = node_fea.astype(jnp.int32).reshape(NW, N_CHUNKS, CHUNK)
    return _emb_lookup(idx2d, table)


# speedup vs baseline: 1.0330x; 1.0002x over previous
import jax
import jax.numpy as jnp
from jax import lax
from jax.experimental import pallas as pl
from jax.experimental.pallas import tpu as pltpu
from jax.experimental.pallas import tpu_sc as plsc

NC, NS, L = 2, 16, 16
NW = NC * NS

B = 16384 * 26
D = 128
CHUNK = 104
B_PER_W = B // NW
N_CHUNKS = B_PER_W // CHUNK
NBUF = 8
N_LOOP = N_CHUNKS // NBUF
TAIL = N_LOOP * NBUF


def _emb_body(table_hbm, idx_hbm, out_hbm, idx_v, bufs, gsems, osems):
    wid = lax.axis_index("s") * NC + lax.axis_index("c")
    row_base = wid * B_PER_W

    pltpu.sync_copy(idx_hbm.at[wid], idx_v)

    def gather(j, b):
        return pltpu.make_async_copy(
            table_hbm.at[idx_v.at[j]], bufs[b], gsems[b])

    def writeback(j, b):
        return pltpu.make_async_copy(
            bufs[b], out_hbm.at[pl.ds(row_base + j * CHUNK, CHUNK)], osems[b])

    for b in range(NBUF):
        gather(b, b).start()

    def step(jj, _):
        for b in range(NBUF):
            j = jj * NBUF + b
            gather(j, b).wait()
            writeback(j, b).start()
            nxt = j + NBUF

            @pl.when(nxt < N_CHUNKS)
            def _():
                writeback(j, b).wait()
                gather(nxt, b).start()
        return 0

    lax.fori_loop(0, N_LOOP, step, 0)

    for j in range(TAIL, N_CHUNKS):
        gather(j, j % NBUF).wait()
        writeback(j, j % NBUF).start()
    for j in range(N_CHUNKS - NBUF, N_CHUNKS):
        writeback(j, j % NBUF).wait()


@jax.jit
def _emb_lookup(idx2d, table):
    mesh = plsc.VectorSubcoreMesh(core_axis_name="c", subcore_axis_name="s")
    f = pl.kernel(
        _emb_body,
        out_type=jax.ShapeDtypeStruct((B, D), jnp.float32),
        mesh=mesh,
        scratch_types=[
            pltpu.VMEM((N_CHUNKS, CHUNK), jnp.int32),
            [pltpu.VMEM((CHUNK, D), jnp.float32) for _ in range(NBUF)],
            [pltpu.SemaphoreType.DMA for _ in range(NBUF)],
            [pltpu.SemaphoreType.DMA for _ in range(NBUF)],
        ],
    )
    return f(table, idx2d)


def kernel(node_fea, table):
    idx2d = node_fea.astype(jnp.int32).reshape(NW, N_CHUNKS, CHUNK)
    return _emb_lookup(idx2d, table)
